# Initial kernel scaffold; baseline (speedup 1.0000x reference)
import jax
import jax.numpy as jnp
from jax.experimental import pallas as pl


def kernel(adj_indices, adj_values, emb, W1_0, b1_0, W2_0, b2_0, W1_1, b1_1, W2_1, b2_1):
    raise NotImplementedError("write your pallas kernel here")


# speedup vs baseline: 4.1110x; 4.1110x over previous
import functools

import jax
import jax.numpy as jnp
from jax import lax
from jax.experimental import pallas as pl
from jax.experimental.pallas import tpu as pltpu
from jax.experimental.pallas import tpu_sc as plsc

NUM_USER = 4000
N = 10000
E = 320000
D = 128

NC = 2
NS = 16
L = 16
NW = NC * NS
EPW = E // NW
C = 80
NCH = EPW // C

def _mesh():
    return plsc.VectorSubcoreMesh(core_axis_name="c", subcore_axis_name="s")


def _bcast_lane(v16, i):
    idx = jnp.full((L,), i, dtype=jnp.int32)
    dnums = lax.GatherDimensionNumbers(
        offset_dims=(), collapsed_slice_dims=(0,), start_index_map=(0,))
    return lax.gather(v16, idx[:, None], dnums, slice_sizes=(1,),
                      mode=lax.GatherScatterMode.PROMISE_IN_BOUNDS)


def _spmm_body(src_h, dst_h, val_h, feat_h, zer_h, out_h,
               si, di, vv, rows, acc, sem):
    c = lax.axis_index("c")
    s = lax.axis_index("s")
    wid = s * NC + c

    @pl.when(s == 0)
    def _():
        pltpu.sync_copy(zer_h, acc)

    plsc.subcore_barrier()

    @pl.loop(0, NCH)
    def _(k):
        base = wid * EPW + k * C
        pltpu.sync_copy(src_h.at[pl.ds(base, C)], si)
        pltpu.sync_copy(dst_h.at[pl.ds(base, C)], di)
        pltpu.sync_copy(val_h.at[pl.ds(base, C)], vv)
        pltpu.async_copy(feat_h.at[si], rows, sem).wait()

        @pl.loop(0, C, step=L)
        def _(r0):
            v16 = vv[pl.ds(r0, L)]
            for i in range(L):
                b = _bcast_lane(v16, i)
                for j in range(D // L):
                    sl = (r0 + i, pl.ds(j * L, L))
                    rows[sl] = rows[sl] * b

        pltpu.sync_copy(rows, acc.at[di], add=True)

    plsc.subcore_barrier()

    @pl.when(s == 0)
    def _():
        pltpu.sync_copy(acc, out_h.at[c])


def _spmm(src, dst, vals, feat, zeros):
    k = functools.partial(
        pl.kernel,
        out_type=jax.ShapeDtypeStruct((2, N, D), jnp.float32),
        mesh=_mesh(),
        scratch_types=[
            pltpu.VMEM((C,), jnp.int32),
            pltpu.VMEM((C,), jnp.int32),
            pltpu.VMEM((C,), jnp.float32),
            pltpu.VMEM((C, D), jnp.float32),
            pltpu.VMEM_SHARED((N, D), jnp.float32),
            pltpu.SemaphoreType.DMA,
        ],
    )(_spmm_body)
    return k(src, dst, vals, feat, zeros)


def _dense_body(s2, x, w1, bb1, w2, bb2, act_o, norm_o):
    side = s2[0] + s2[1]
    xv = x[...]
    su = side + xv
    bi = side * xv
    out = (jnp.dot(su, w1[...], preferred_element_type=jnp.float32)
           + jnp.dot(bi, w2[...], preferred_element_type=jnp.float32)
           + bb1[...] + bb2[...])
    act = jnp.where(out >= 0.0, out, 0.01 * out)
    act_o[...] = act
    nrm = jnp.sqrt(jnp.sum(act * act, axis=-1, keepdims=True))
    norm_o[...] = act / jnp.maximum(nrm, 1e-12)


def _dense(side2, nf, W1, b1, W2, b2):
    BR = 1000
    return pl.pallas_call(
        _dense_body,
        grid=(N // BR,),
        in_specs=[
            pl.BlockSpec((2, BR, D), lambda i: (0, i, 0)),
            pl.BlockSpec((BR, D), lambda i: (i, 0)),
            pl.BlockSpec((D, D), lambda i: (0, 0)),
            pl.BlockSpec((1, D), lambda i: (0, 0)),
            pl.BlockSpec((D, D), lambda i: (0, 0)),
            pl.BlockSpec((1, D), lambda i: (0, 0)),
        ],
        out_specs=[pl.BlockSpec((BR, D), lambda i: (i, 0)),
                   pl.BlockSpec((BR, D), lambda i: (i, 0))],
        out_shape=[jax.ShapeDtypeStruct((N, D), jnp.float32),
                   jax.ShapeDtypeStruct((N, D), jnp.float32)],
    )(side2, nf, W1, b1.reshape(1, D), W2, b2.reshape(1, D))


def kernel(adj_indices, adj_values, emb,
           W1_0, b1_0, W2_0, b2_0, W1_1, b1_1, W2_1, b2_1):
    src = adj_indices[0]
    dst = adj_indices[1]
    zeros = jnp.zeros((N, D), jnp.float32)
    side_p1 = _spmm(src, dst, adj_values, emb, zeros)
    nf1, norm1 = _dense(side_p1, emb, W1_0, b1_0, W2_0, b2_0)
    side_p2 = _spmm(src, dst, adj_values, nf1, zeros)
    nf2, norm2 = _dense(side_p2, nf1, W1_1, b1_1, W2_1, b2_1)
    ew = jnp.concatenate([emb, norm1, norm2], axis=-1)
    return ew[:NUM_USER], ew[NUM_USER:]


# speedup vs baseline: 10.6651x; 2.5943x over previous
import dataclasses
import functools

import jax
import jax.numpy as jnp
from jax import lax
from jax.experimental import pallas as pl
from jax.experimental.pallas import tpu as pltpu
from jax.experimental.pallas import tpu_sc as plsc

NUM_USER = 4000
N = 10000
E = 320000
D = 128

NC = 2
NS = 16
L = 16
NW = NC * NS
EPW = E // NW
C = 80
NCH = EPW // C

NROWS = 4
NPB = 8


def _mesh():
    return plsc.VectorSubcoreMesh(core_axis_name="c", subcore_axis_name="s")


def _sc_params():
    cp = pltpu.CompilerParams()
    if "needs_layout_passes" in pltpu.CompilerParams.__dataclass_fields__:
        cp = dataclasses.replace(cp, needs_layout_passes=False)
    return cp


def _bcast_lane(v16, i):
    idx = jnp.full((L,), i, dtype=jnp.int32)
    dnums = lax.GatherDimensionNumbers(
        offset_dims=(), collapsed_slice_dims=(0,), start_index_map=(0,))
    return lax.gather(v16, idx[:, None], dnums, slice_sizes=(1,),
                      mode=lax.GatherScatterMode.PROMISE_IN_BOUNDS)


def _spmm_body(pall_h, val_h, feat_h, zer_h, out_h,
               pbuf, vbuf, rows, acc, sg, ss, sp, sv):
    c = lax.axis_index("c")
    s = lax.axis_index("s")
    wid = s * NC + c

    @pl.when(s == 0)
    def _():
        pltpu.sync_copy(zer_h, acc)

    plsc.subcore_barrier()

    def pdma_start(k, j):
        pltpu.async_copy(pall_h.at[wid, k], pbuf.at[j], sp.at[j])
        pltpu.async_copy(val_h.at[pl.ds(wid * EPW + k * C, C)],
                         vbuf.at[pl.ds(j * C, C)], sv.at[j])

    def pdma_wait(k, j):
        pltpu.make_async_copy(pall_h.at[wid, k], pbuf.at[j], sp.at[j]).wait()
        pltpu.make_async_copy(val_h.at[pl.ds(wid * EPW + k * C, C)],
                              vbuf.at[pl.ds(j * C, C)], sv.at[j]).wait()

    def gather_start(b, j):
        pltpu.async_copy(feat_h.at[pbuf.at[j, 0]], rows.at[b], sg.at[b])

    def gather_wait(b, j):
        pltpu.make_async_copy(feat_h.at[pbuf.at[j, 0]], rows.at[b],
                              sg.at[b]).wait()

    def scat_start(b, j):
        pltpu.async_copy(rows.at[b], acc.at[pbuf.at[j, 1]], ss.at[b],
                         add=True)

    def scat_wait(b, j):
        pltpu.make_async_copy(rows.at[b], acc.at[pbuf.at[j, 1]],
                              ss.at[b]).wait()

    def scale(b, j):
        @pl.loop(0, C, step=L)
        def _(r0):
            v16 = vbuf[pl.ds(j * C + r0, L)]
            for i in range(L):
                bc = _bcast_lane(v16, i)
                for q in range(D // L):
                    sl = (b, r0 + i, pl.ds(q * L, L))
                    rows[sl] = rows[sl] * bc

    def visit(v, u, in_prologue=False, guard_pdma=False):
        b = u % NROWS
        j = u % NPB
        b2 = (u + 2) % NROWS
        j6 = (u + 6) % NPB
        j2 = (u + 2) % NPB
        gather_wait(b, j)
        scale(b, j)
        scat_start(b, j)
        if not (in_prologue and u < 2):
            scat_wait(b2, j6)
        if guard_pdma:
            @pl.when(v < NCH - 6)
            def _():
                pdma_start(v + 6, j6)
        else:
            pdma_start(v + 6, j6)
        pdma_wait(v + 2, j2)
        gather_start(b2, j2)

    for j in range(6):
        pdma_start(j, j)
    pdma_wait(0, 0)
    gather_start(0, 0)
    pdma_wait(1, 1)
    gather_start(1, 1)
    for v in range(8):
        visit(v, v, in_prologue=True)

    @pl.loop(8, 120, step=8)
    def _(v0):
        for u in range(8):
            visit(v0 + u, u, guard_pdma=True)

    for v in range(120, 125):
        u = v % 8
        b = u % NROWS
        j = u % NPB
        b2 = (u + 2) % NROWS
        j6 = (u + 6) % NPB
        j2 = (u + 2) % NPB
        gather_wait(b, j)
        scale(b, j)
        scat_start(b, j)
        scat_wait(b2, j6)
        if v + 2 <= 124:
            pdma_wait(v + 2, j2)
            gather_start(b2, j2)
    scat_wait(123 % NROWS, 123 % NPB)
    scat_wait(124 % NROWS, 124 % NPB)

    plsc.subcore_barrier()

    @pl.when(s == 0)
    def _():
        pltpu.sync_copy(acc, out_h.at[c])


def _spmm(pall, vals, feat, zeros):
    k = functools.partial(
        pl.kernel,
        out_type=jax.ShapeDtypeStruct((2, N, D), jnp.float32),
        mesh=_mesh(),
        compiler_params=_sc_params(),
        scratch_types=[
            pltpu.VMEM((NPB, 2, C), jnp.int32),
            pltpu.VMEM((NPB * C,), jnp.float32),
            pltpu.VMEM((NROWS, C, D), jnp.float32),
            pltpu.VMEM_SHARED((N, D), jnp.float32),
            pltpu.SemaphoreType.DMA((NROWS,)),
            pltpu.SemaphoreType.DMA((NROWS,)),
            pltpu.SemaphoreType.DMA((NPB,)),
            pltpu.SemaphoreType.DMA((NPB,)),
        ],
    )(_spmm_body)
    return k(pall, vals, feat, zeros)


def _dense_body(s2, x, w1, bb1, w2, bb2, act_o, norm_o):
    side = s2[0] + s2[1]
    xv = x[...]
    su = side + xv
    bi = side * xv
    out = (jnp.dot(su, w1[...], preferred_element_type=jnp.float32)
           + jnp.dot(bi, w2[...], preferred_element_type=jnp.float32)
           + bb1[...] + bb2[...])
    act = jnp.where(out >= 0.0, out, 0.01 * out)
    act_o[...] = act
    nrm = jnp.sqrt(jnp.sum(act * act, axis=-1, keepdims=True))
    norm_o[...] = act / jnp.maximum(nrm, 1e-12)


def _dense(side2, nf, W1, b1, W2, b2):
    BR = 1000
    return pl.pallas_call(
        _dense_body,
        grid=(N // BR,),
        in_specs=[
            pl.BlockSpec((2, BR, D), lambda i: (0, i, 0)),
            pl.BlockSpec((BR, D), lambda i: (i, 0)),
            pl.BlockSpec((D, D), lambda i: (0, 0)),
            pl.BlockSpec((1, D), lambda i: (0, 0)),
            pl.BlockSpec((D, D), lambda i: (0, 0)),
            pl.BlockSpec((1, D), lambda i: (0, 0)),
        ],
        out_specs=[pl.BlockSpec((BR, D), lambda i: (i, 0)),
                   pl.BlockSpec((BR, D), lambda i: (i, 0))],
        out_shape=[jax.ShapeDtypeStruct((N, D), jnp.float32),
                   jax.ShapeDtypeStruct((N, D), jnp.float32)],
    )(side2, nf, W1, b1.reshape(1, D), W2, b2.reshape(1, D))


def kernel(adj_indices, adj_values, emb,
           W1_0, b1_0, W2_0, b2_0, W1_1, b1_1, W2_1, b2_1):
    src = adj_indices[0].reshape(NW, NCH, C)
    dst = adj_indices[1].reshape(NW, NCH, C)
    pall = jnp.stack([src, dst], axis=2)
    zeros = jnp.zeros((N, D), jnp.float32)
    side_p1 = _spmm(pall, adj_values, emb, zeros)
    nf1, norm1 = _dense(side_p1, emb, W1_0, b1_0, W2_0, b2_0)
    side_p2 = _spmm(pall, adj_values, nf1, zeros)
    nf2, norm2 = _dense(side_p2, nf1, W1_1, b1_1, W2_1, b2_1)
    ew = jnp.concatenate([emb, norm1, norm2], axis=-1)
    return ew[:NUM_USER], ew[NUM_USER:]


# speedup vs baseline: 10.9349x; 1.0253x over previous
import dataclasses
import functools

import jax
import jax.numpy as jnp
from jax import lax
from jax.experimental import pallas as pl
from jax.experimental.pallas import tpu as pltpu
from jax.experimental.pallas import tpu_sc as plsc

NUM_USER = 4000
N = 10000
E = 320000
D = 128

NC = 2
NS = 16
L = 16
NW = NC * NS
EPW = E // NW
C = 80
NCH = EPW // C

NROWS = 4
NPB = 8


def _mesh():
    return plsc.VectorSubcoreMesh(core_axis_name="c", subcore_axis_name="s")


def _sc_params():
    cp = pltpu.CompilerParams()
    if "needs_layout_passes" in pltpu.CompilerParams.__dataclass_fields__:
        cp = dataclasses.replace(cp, needs_layout_passes=False)
    return cp


def _bcast_lane(v16, i):
    idx = jnp.full((L,), i, dtype=jnp.int32)
    dnums = lax.GatherDimensionNumbers(
        offset_dims=(), collapsed_slice_dims=(0,), start_index_map=(0,))
    return lax.gather(v16, idx[:, None], dnums, slice_sizes=(1,),
                      mode=lax.GatherScatterMode.PROMISE_IN_BOUNDS)


def _spmm_body(pall_h, val_h, feat_h, out_h,
               pbuf, vbuf, rows, zbuf, acc, sg, ss, sp, sv, sz):
    c = lax.axis_index("c")
    s = lax.axis_index("s")
    wid = s * NC + c

    zbase = s * 624
    @pl.loop(0, 32)
    def _(r):
        z = jnp.zeros((L,), jnp.float32)
        for q in range(D // L):
            zbuf[r, pl.ds(q * L, L)] = z
    for t in range(20):
        pltpu.async_copy(zbuf, acc.at[pl.ds(zbase + t * 32, 32)], sz)
    for t in range(20):
        pltpu.make_async_copy(zbuf, acc.at[pl.ds(zbase + t * 32, 32)],
                              sz).wait()

    plsc.subcore_barrier()

    def pdma_start(k, j):
        pltpu.async_copy(pall_h.at[wid, k], pbuf.at[j], sp.at[j])
        pltpu.async_copy(val_h.at[pl.ds(wid * EPW + k * C, C)],
                         vbuf.at[pl.ds(j * C, C)], sv.at[j])

    def pdma_wait(k, j):
        pltpu.make_async_copy(pall_h.at[wid, k], pbuf.at[j], sp.at[j]).wait()
        pltpu.make_async_copy(val_h.at[pl.ds(wid * EPW + k * C, C)],
                              vbuf.at[pl.ds(j * C, C)], sv.at[j]).wait()

    def gather_start(b, j):
        pltpu.async_copy(feat_h.at[pbuf.at[j, 0]], rows.at[b], sg.at[b])

    def gather_wait(b, j):
        pltpu.make_async_copy(feat_h.at[pbuf.at[j, 0]], rows.at[b],
                              sg.at[b]).wait()

    def scat_start(b, j):
        pltpu.async_copy(rows.at[b], acc.at[pbuf.at[j, 1]], ss.at[b],
                         add=True)

    def scat_wait(b, j):
        pltpu.make_async_copy(rows.at[b], acc.at[pbuf.at[j, 1]],
                              ss.at[b]).wait()

    def scale(b, j):
        @pl.loop(0, C, step=L)
        def _(r0):
            v16 = vbuf[pl.ds(j * C + r0, L)]
            for i in range(L):
                bc = _bcast_lane(v16, i)
                for q in range(D // L):
                    sl = (b, r0 + i, pl.ds(q * L, L))
                    rows[sl] = rows[sl] * bc

    def visit(v, u, in_prologue=False, guard_pdma=False):
        b = u % NROWS
        j = u % NPB
        b2 = (u + 2) % NROWS
        j6 = (u + 6) % NPB
        j2 = (u + 2) % NPB
        gather_wait(b, j)
        scale(b, j)
        scat_start(b, j)
        if not (in_prologue and u < 2):
            scat_wait(b2, j6)
        if guard_pdma:
            @pl.when(v < NCH - 6)
            def _():
                pdma_start(v + 6, j6)
        else:
            pdma_start(v + 6, j6)
        pdma_wait(v + 2, j2)
        gather_start(b2, j2)

    for j in range(6):
        pdma_start(j, j)
    pdma_wait(0, 0)
    gather_start(0, 0)
    pdma_wait(1, 1)
    gather_start(1, 1)
    for v in range(8):
        visit(v, v, in_prologue=True)

    @pl.loop(8, 120, step=8)
    def _(v0):
        for u in range(8):
            visit(v0 + u, u, guard_pdma=True)

    for v in range(120, 125):
        u = v % 8
        b = u % NROWS
        j = u % NPB
        b2 = (u + 2) % NROWS
        j6 = (u + 6) % NPB
        j2 = (u + 2) % NPB
        gather_wait(b, j)
        scale(b, j)
        scat_start(b, j)
        scat_wait(b2, j6)
        if v + 2 <= 124:
            pdma_wait(v + 2, j2)
            gather_start(b2, j2)
    scat_wait(123 % NROWS, 123 % NPB)
    scat_wait(124 % NROWS, 124 % NPB)

    plsc.subcore_barrier()

    base = s * 624
    pltpu.sync_copy(acc.at[pl.ds(base, 640)],
                    out_h.at[c, pl.ds(base, 640)])


def _spmm(pall, vals, feat):
    k = functools.partial(
        pl.kernel,
        out_type=jax.ShapeDtypeStruct((2, N, D), jnp.float32),
        mesh=_mesh(),
        compiler_params=_sc_params(),
        scratch_types=[
            pltpu.VMEM((NPB, 2, C), jnp.int32),
            pltpu.VMEM((NPB * C,), jnp.float32),
            pltpu.VMEM((NROWS, C, D), jnp.float32),
            pltpu.VMEM((32, D), jnp.float32),
            pltpu.VMEM_SHARED((N, D), jnp.float32),
            pltpu.SemaphoreType.DMA((NROWS,)),
            pltpu.SemaphoreType.DMA((NROWS,)),
            pltpu.SemaphoreType.DMA((NPB,)),
            pltpu.SemaphoreType.DMA((NPB,)),
            pltpu.SemaphoreType.DMA,
        ],
    )(_spmm_body)
    return k(pall, vals, feat)


def _dense_body(s2, x, w1, bb1, w2, bb2, act_o, norm_o):
    side = s2[0] + s2[1]
    xv = x[...]
    su = side + xv
    bi = side * xv
    out = (jnp.dot(su, w1[...], preferred_element_type=jnp.float32)
           + jnp.dot(bi, w2[...], preferred_element_type=jnp.float32)
           + bb1[...] + bb2[...])
    act = jnp.where(out >= 0.0, out, 0.01 * out)
    act_o[...] = act
    nrm = jnp.sqrt(jnp.sum(act * act, axis=-1, keepdims=True))
    norm_o[...] = act / jnp.maximum(nrm, 1e-12)


def _dense(side2, nf, W1, b1, W2, b2):
    BR = 1000
    return pl.pallas_call(
        _dense_body,
        grid=(N // BR,),
        in_specs=[
            pl.BlockSpec((2, BR, D), lambda i: (0, i, 0)),
            pl.BlockSpec((BR, D), lambda i: (i, 0)),
            pl.BlockSpec((D, D), lambda i: (0, 0)),
            pl.BlockSpec((1, D), lambda i: (0, 0)),
            pl.BlockSpec((D, D), lambda i: (0, 0)),
            pl.BlockSpec((1, D), lambda i: (0, 0)),
        ],
        out_specs=[pl.BlockSpec((BR, D), lambda i: (i, 0)),
                   pl.BlockSpec((BR, D), lambda i: (i, 0))],
        out_shape=[jax.ShapeDtypeStruct((N, D), jnp.float32),
                   jax.ShapeDtypeStruct((N, D), jnp.float32)],
    )(side2, nf, W1, b1.reshape(1, D), W2, b2.reshape(1, D))


def kernel(adj_indices, adj_values, emb,
           W1_0, b1_0, W2_0, b2_0, W1_1, b1_1, W2_1, b2_1):
    src = adj_indices[0].reshape(NW, NCH, C)
    dst = adj_indices[1].reshape(NW, NCH, C)
    pall = jnp.stack([src, dst], axis=2)
    side_p1 = _spmm(pall, adj_values, emb)
    nf1, norm1 = _dense(side_p1, emb, W1_0, b1_0, W2_0, b2_0)
    side_p2 = _spmm(pall, adj_values, nf1)
    nf2, norm2 = _dense(side_p2, nf1, W1_1, b1_1, W2_1, b2_1)
    ew = jnp.concatenate([emb, norm1, norm2], axis=-1)
    return ew[:NUM_USER], ew[NUM_USER:]


# speedup vs baseline: 11.7961x; 1.0788x over previous
import dataclasses
import functools

import jax
import jax.numpy as jnp
from jax import lax
from jax.experimental import pallas as pl
from jax.experimental.pallas import tpu as pltpu
from jax.experimental.pallas import tpu_sc as plsc

NUM_USER = 4000
N = 10000
E = 320000
D = 128

NC = 2
NS = 16
L = 16
NW = NC * NS
EPW = E // NW
C = 80
NCH = EPW // C

NROWS = 4
NPB = 8


def _mesh():
    return plsc.VectorSubcoreMesh(core_axis_name="c", subcore_axis_name="s")


def _sc_params():
    cp = pltpu.CompilerParams()
    if "needs_layout_passes" in pltpu.CompilerParams.__dataclass_fields__:
        cp = dataclasses.replace(cp, needs_layout_passes=False)
    return cp


def _bcast_lane(v16, i):
    idx = jnp.full((L,), i, dtype=jnp.int32)
    dnums = lax.GatherDimensionNumbers(
        offset_dims=(), collapsed_slice_dims=(0,), start_index_map=(0,))
    return lax.gather(v16, idx[:, None], dnums, slice_sizes=(1,),
                      mode=lax.GatherScatterMode.PROMISE_IN_BOUNDS)


def _spmm_body(idx_h, val_h, feat_h, out_h,
               pbuf, vbuf, rows, zbuf, acc, sg, ss, sp, sv, sz):
    c = lax.axis_index("c")
    s = lax.axis_index("s")
    wid = s * NC + c

    zbase = s * 624
    @pl.loop(0, 32)
    def _(r):
        z = jnp.zeros((L,), jnp.float32)
        for q in range(D // L):
            zbuf[r, pl.ds(q * L, L)] = z
    for t in range(20):
        pltpu.async_copy(zbuf, acc.at[pl.ds(zbase + t * 32, 32)], sz)
    for t in range(20):
        pltpu.make_async_copy(zbuf, acc.at[pl.ds(zbase + t * 32, 32)],
                              sz).wait()

    plsc.subcore_barrier()

    def pdma_start(k, j):
        off = wid * EPW + k * C
        pltpu.async_copy(idx_h.at[pl.ds(off, C)], pbuf.at[j, 0], sp.at[j])
        pltpu.async_copy(idx_h.at[pl.ds(E + off, C)], pbuf.at[j, 1],
                         sp.at[j])
        pltpu.async_copy(val_h.at[pl.ds(off, C)],
                         vbuf.at[pl.ds(j * C, C)], sv.at[j])

    def pdma_wait(k, j):
        off = wid * EPW + k * C
        pltpu.make_async_copy(idx_h.at[pl.ds(off, C)], pbuf.at[j, 0],
                              sp.at[j]).wait()
        pltpu.make_async_copy(idx_h.at[pl.ds(E + off, C)], pbuf.at[j, 1],
                              sp.at[j]).wait()
        pltpu.make_async_copy(val_h.at[pl.ds(off, C)],
                              vbuf.at[pl.ds(j * C, C)], sv.at[j]).wait()

    def gather_start(b, j):
        pltpu.async_copy(feat_h.at[pbuf.at[j, 0]], rows.at[b], sg.at[b])

    def gather_wait(b, j):
        pltpu.make_async_copy(feat_h.at[pbuf.at[j, 0]], rows.at[b],
                              sg.at[b]).wait()

    def scat_start(b, j):
        pltpu.async_copy(rows.at[b], acc.at[pbuf.at[j, 1]], ss.at[b],
                         add=True)

    def scat_wait(b, j):
        pltpu.make_async_copy(rows.at[b], acc.at[pbuf.at[j, 1]],
                              ss.at[b]).wait()

    def scale(b, j):
        @pl.loop(0, C, step=L)
        def _(r0):
            v16 = vbuf[pl.ds(j * C + r0, L)]
            for i in range(L):
                bc = _bcast_lane(v16, i)
                for q in range(D // L):
                    sl = (b, r0 + i, pl.ds(q * L, L))
                    rows[sl] = rows[sl] * bc

    def visit(v, u, in_prologue=False, guard_pdma=False):
        b = u % NROWS
        j = u % NPB
        b2 = (u + 2) % NROWS
        j6 = (u + 6) % NPB
        j2 = (u + 2) % NPB
        gather_wait(b, j)
        scale(b, j)
        scat_start(b, j)
        if not (in_prologue and u < 2):
            scat_wait(b2, j6)
        if guard_pdma:
            @pl.when(v < NCH - 6)
            def _():
                pdma_start(v + 6, j6)
        else:
            pdma_start(v + 6, j6)
        pdma_wait(v + 2, j2)
        gather_start(b2, j2)

    for j in range(6):
        pdma_start(j, j)
    pdma_wait(0, 0)
    gather_start(0, 0)
    pdma_wait(1, 1)
    gather_start(1, 1)
    for v in range(8):
        visit(v, v, in_prologue=True)

    @pl.loop(8, 120, step=8)
    def _(v0):
        for u in range(8):
            visit(v0 + u, u, guard_pdma=True)

    for v in range(120, 125):
        u = v % 8
        b = u % NROWS
        j = u % NPB
        b2 = (u + 2) % NROWS
        j6 = (u + 6) % NPB
        j2 = (u + 2) % NPB
        gather_wait(b, j)
        scale(b, j)
        scat_start(b, j)
        scat_wait(b2, j6)
        if v + 2 <= 124:
            pdma_wait(v + 2, j2)
            gather_start(b2, j2)
    scat_wait(123 % NROWS, 123 % NPB)
    scat_wait(124 % NROWS, 124 % NPB)

    plsc.subcore_barrier()

    base = s * 624
    pltpu.sync_copy(acc.at[pl.ds(base, 640)],
                    out_h.at[c, pl.ds(base, 640)])


def _spmm(idx, vals, feat):
    k = functools.partial(
        pl.kernel,
        out_type=jax.ShapeDtypeStruct((2, N, D), jnp.float32),
        mesh=_mesh(),
        compiler_params=_sc_params(),
        scratch_types=[
            pltpu.VMEM((NPB, 2, C), jnp.int32),
            pltpu.VMEM((NPB * C,), jnp.float32),
            pltpu.VMEM((NROWS, C, D), jnp.float32),
            pltpu.VMEM((32, D), jnp.float32),
            pltpu.VMEM_SHARED((N, D), jnp.float32),
            pltpu.SemaphoreType.DMA((NROWS,)),
            pltpu.SemaphoreType.DMA((NROWS,)),
            pltpu.SemaphoreType.DMA((NPB,)),
            pltpu.SemaphoreType.DMA((NPB,)),
            pltpu.SemaphoreType.DMA,
        ],
    )(_spmm_body)
    return k(idx, vals, feat)


def _dense_body(s2, x, w1, bb1, w2, bb2, act_o, norm_o):
    side = s2[0] + s2[1]
    xv = x[...]
    su = side + xv
    bi = side * xv
    out = (jnp.dot(su, w1[...], preferred_element_type=jnp.float32)
           + jnp.dot(bi, w2[...], preferred_element_type=jnp.float32)
           + bb1[...] + bb2[...])
    act = jnp.where(out >= 0.0, out, 0.01 * out)
    act_o[...] = act
    nrm = jnp.sqrt(jnp.sum(act * act, axis=-1, keepdims=True))
    norm_o[...] = act / jnp.maximum(nrm, 1e-12)


def _dense(side2, nf, W1, b1, W2, b2):
    BR = 1000
    return pl.pallas_call(
        _dense_body,
        grid=(N // BR,),
        in_specs=[
            pl.BlockSpec((2, BR, D), lambda i: (0, i, 0)),
            pl.BlockSpec((BR, D), lambda i: (i, 0)),
            pl.BlockSpec((D, D), lambda i: (0, 0)),
            pl.BlockSpec((1, D), lambda i: (0, 0)),
            pl.BlockSpec((D, D), lambda i: (0, 0)),
            pl.BlockSpec((1, D), lambda i: (0, 0)),
        ],
        out_specs=[pl.BlockSpec((BR, D), lambda i: (i, 0)),
                   pl.BlockSpec((BR, D), lambda i: (i, 0))],
        out_shape=[jax.ShapeDtypeStruct((N, D), jnp.float32),
                   jax.ShapeDtypeStruct((N, D), jnp.float32)],
    )(side2, nf, W1, b1.reshape(1, D), W2, b2.reshape(1, D))


def kernel(adj_indices, adj_values, emb,
           W1_0, b1_0, W2_0, b2_0, W1_1, b1_1, W2_1, b2_1):
    idxf = adj_indices.reshape(2 * E)
    side_p1 = _spmm(idxf, adj_values, emb)
    nf1, norm1 = _dense(side_p1, emb, W1_0, b1_0, W2_0, b2_0)
    side_p2 = _spmm(idxf, adj_values, nf1)
    nf2, norm2 = _dense(side_p2, nf1, W1_1, b1_1, W2_1, b2_1)
    ew = jnp.concatenate([emb, norm1, norm2], axis=-1)
    return ew[:NUM_USER], ew[NUM_USER:]


# speedup vs baseline: 12.2912x; 1.0420x over previous
import dataclasses
import functools

import jax
import jax.numpy as jnp
from jax import lax
from jax.experimental import pallas as pl
from jax.experimental.pallas import tpu as pltpu
from jax.experimental.pallas import tpu_sc as plsc

NUM_USER = 4000
N = 10000
E = 320000
D = 128

NC = 2
NS = 16
L = 16
NW = NC * NS
EPW = E // NW
C = 80
NCH = EPW // C

NROWS = 4
NPB = 8


def _mesh():
    return plsc.VectorSubcoreMesh(core_axis_name="c", subcore_axis_name="s")


def _sc_params():
    cp = pltpu.CompilerParams()
    if "needs_layout_passes" in pltpu.CompilerParams.__dataclass_fields__:
        cp = dataclasses.replace(cp, needs_layout_passes=False)
    return cp


def _bcast_lane(v16, i):
    idx = jnp.full((L,), i, dtype=jnp.int32)
    dnums = lax.GatherDimensionNumbers(
        offset_dims=(), collapsed_slice_dims=(0,), start_index_map=(0,))
    return lax.gather(v16, idx[:, None], dnums, slice_sizes=(1,),
                      mode=lax.GatherScatterMode.PROMISE_IN_BOUNDS)


def _spmm_body(idx_h, val_h, feat_h, out_h,
               pbuf, vbuf, rows, zbuf, acc, sg, ss, sp, sv, sz):
    c = lax.axis_index("c")
    s = lax.axis_index("s")
    wid = s * NC + c

    zbase = s * 624
    @pl.loop(0, 32)
    def _(r):
        z = jnp.zeros((L,), jnp.float32)
        for q in range(D // L):
            zbuf[r, pl.ds(q * L, L)] = z
    for t in range(20):
        pltpu.async_copy(zbuf, acc.at[pl.ds(zbase + t * 32, 32)], sz)
    for t in range(20):
        pltpu.make_async_copy(zbuf, acc.at[pl.ds(zbase + t * 32, 32)],
                              sz).wait()

    plsc.subcore_barrier()

    def pdma_start(k, j):
        off = wid * EPW + k * C
        pltpu.async_copy(idx_h.at[pl.ds(off, C)], pbuf.at[j, 0], sp.at[j])
        pltpu.async_copy(idx_h.at[pl.ds(E + off, C)], pbuf.at[j, 1],
                         sp.at[j])
        pltpu.async_copy(val_h.at[pl.ds(off, C)],
                         vbuf.at[pl.ds(j * C, C)], sv.at[j])

    def pdma_wait(k, j):
        off = wid * EPW + k * C
        pltpu.make_async_copy(idx_h.at[pl.ds(off, C)], pbuf.at[j, 0],
                              sp.at[j]).wait()
        pltpu.make_async_copy(idx_h.at[pl.ds(E + off, C)], pbuf.at[j, 1],
                              sp.at[j]).wait()
        pltpu.make_async_copy(val_h.at[pl.ds(off, C)],
                              vbuf.at[pl.ds(j * C, C)], sv.at[j]).wait()

    def gather_start(b, j):
        pltpu.async_copy(feat_h.at[pbuf.at[j, 0]], rows.at[b], sg.at[b])

    def gather_wait(b, j):
        pltpu.make_async_copy(feat_h.at[pbuf.at[j, 0]], rows.at[b],
                              sg.at[b]).wait()

    def scat_start(b, j):
        pltpu.async_copy(rows.at[b], acc.at[pbuf.at[j, 1]], ss.at[b],
                         add=True)

    def scat_wait(b, j):
        pltpu.make_async_copy(rows.at[b], acc.at[pbuf.at[j, 1]],
                              ss.at[b]).wait()

    def scale(b, j):
        @pl.loop(0, C, step=L)
        def _(r0):
            v16 = vbuf[pl.ds(j * C + r0, L)]
            for i in range(L):
                bc = _bcast_lane(v16, i)
                for q in range(D // L):
                    sl = (b, r0 + i, pl.ds(q * L, L))
                    rows[sl] = rows[sl] * bc

    def visit(v, u, in_prologue=False, guard_pdma=False):
        b = u % NROWS
        j = u % NPB
        b2 = (u + 2) % NROWS
        j6 = (u + 6) % NPB
        j2 = (u + 2) % NPB
        gather_wait(b, j)
        scale(b, j)
        scat_start(b, j)
        if not (in_prologue and u < 2):
            scat_wait(b2, j6)
        if guard_pdma:
            @pl.when(v < NCH - 6)
            def _():
                pdma_start(v + 6, j6)
        else:
            pdma_start(v + 6, j6)
        pdma_wait(v + 2, j2)
        gather_start(b2, j2)

    for j in range(6):
        pdma_start(j, j)
    pdma_wait(0, 0)
    gather_start(0, 0)
    pdma_wait(1, 1)
    gather_start(1, 1)
    for v in range(8):
        visit(v, v, in_prologue=True)

    @pl.loop(8, 120, step=8)
    def _(v0):
        for u in range(8):
            visit(v0 + u, u, guard_pdma=True)

    for v in range(120, 125):
        u = v % 8
        b = u % NROWS
        j = u % NPB
        b2 = (u + 2) % NROWS
        j6 = (u + 6) % NPB
        j2 = (u + 2) % NPB
        gather_wait(b, j)
        scale(b, j)
        scat_start(b, j)
        scat_wait(b2, j6)
        if v + 2 <= 124:
            pdma_wait(v + 2, j2)
            gather_start(b2, j2)
    scat_wait(123 % NROWS, 123 % NPB)
    scat_wait(124 % NROWS, 124 % NPB)

    plsc.subcore_barrier()

    base = s * 624
    pltpu.sync_copy(acc.at[pl.ds(base, 640)],
                    out_h.at[c, pl.ds(base, 640)])


def _spmm(idx, vals, feat):
    k = functools.partial(
        pl.kernel,
        out_type=jax.ShapeDtypeStruct((2, N, D), jnp.float32),
        mesh=_mesh(),
        compiler_params=_sc_params(),
        scratch_types=[
            pltpu.VMEM((NPB, 2, C), jnp.int32),
            pltpu.VMEM((NPB * C,), jnp.float32),
            pltpu.VMEM((NROWS, C, D), jnp.float32),
            pltpu.VMEM((32, D), jnp.float32),
            pltpu.VMEM_SHARED((N, D), jnp.float32),
            pltpu.SemaphoreType.DMA((NROWS,)),
            pltpu.SemaphoreType.DMA((NROWS,)),
            pltpu.SemaphoreType.DMA((NPB,)),
            pltpu.SemaphoreType.DMA((NPB,)),
            pltpu.SemaphoreType.DMA,
        ],
    )(_spmm_body)
    return k(idx, vals, feat)


def _dense_body(s2, x, w1, bb1, w2, bb2, act_o, norm_o):
    side = s2[0] + s2[1]
    xv = x[...]
    su = side + xv
    bi = side * xv
    out = (jnp.dot(su, w1[...], preferred_element_type=jnp.float32)
           + jnp.dot(bi, w2[...], preferred_element_type=jnp.float32)
           + bb1[...] + bb2[...])
    act = jnp.where(out >= 0.0, out, 0.01 * out)
    act_o[...] = act
    nrm = jnp.sqrt(jnp.sum(act * act, axis=-1, keepdims=True))
    norm_o[...] = act / jnp.maximum(nrm, 1e-12)


def _dense(side2, nf, W1, b1, W2, b2):
    BR = 1000
    return pl.pallas_call(
        _dense_body,
        grid=(N // BR,),
        in_specs=[
            pl.BlockSpec((2, BR, D), lambda i: (0, i, 0)),
            pl.BlockSpec((BR, D), lambda i: (i, 0)),
            pl.BlockSpec((D, D), lambda i: (0, 0)),
            pl.BlockSpec((1, D), lambda i: (0, 0)),
            pl.BlockSpec((D, D), lambda i: (0, 0)),
            pl.BlockSpec((1, D), lambda i: (0, 0)),
        ],
        out_specs=[pl.BlockSpec((BR, D), lambda i: (i, 0)),
                   pl.BlockSpec((BR, D), lambda i: (i, 0))],
        out_shape=[jax.ShapeDtypeStruct((N, D), jnp.float32),
                   jax.ShapeDtypeStruct((N, D), jnp.float32)],
    )(side2, nf, W1, b1.reshape(1, D), W2, b2.reshape(1, D))


def _dense_final_body(s2, x, w1, bb1, w2, bb2, embr, n1r, uw_o, iw_o):
    i = pl.program_id(0)
    side = s2[0] + s2[1]
    xv = x[...]
    su = side + xv
    bi = side * xv
    out = (jnp.dot(su, w1[...], preferred_element_type=jnp.float32)
           + jnp.dot(bi, w2[...], preferred_element_type=jnp.float32)
           + bb1[...] + bb2[...])
    act = jnp.where(out >= 0.0, out, 0.01 * out)
    nrm = jnp.sqrt(jnp.sum(act * act, axis=-1, keepdims=True))
    norm = act / jnp.maximum(nrm, 1e-12)
    full = jnp.concatenate([embr[...], n1r[...], norm], axis=-1)

    @pl.when(i < NUM_USER // 1000)
    def _():
        uw_o[...] = full

    @pl.when(i >= NUM_USER // 1000)
    def _():
        iw_o[...] = full


def _dense_final(side2, nf, W1, b1, W2, b2, emb, norm1):
    BR = 1000
    nu = NUM_USER // BR
    return pl.pallas_call(
        _dense_final_body,
        grid=(N // BR,),
        in_specs=[
            pl.BlockSpec((2, BR, D), lambda i: (0, i, 0)),
            pl.BlockSpec((BR, D), lambda i: (i, 0)),
            pl.BlockSpec((D, D), lambda i: (0, 0)),
            pl.BlockSpec((1, D), lambda i: (0, 0)),
            pl.BlockSpec((D, D), lambda i: (0, 0)),
            pl.BlockSpec((1, D), lambda i: (0, 0)),
            pl.BlockSpec((BR, D), lambda i: (i, 0)),
            pl.BlockSpec((BR, D), lambda i: (i, 0)),
        ],
        out_specs=[
            pl.BlockSpec((BR, 3 * D), lambda i: (jnp.minimum(i, nu - 1), 0)),
            pl.BlockSpec((BR, 3 * D), lambda i: (jnp.maximum(i - nu, 0), 0)),
        ],
        out_shape=[jax.ShapeDtypeStruct((NUM_USER, 3 * D), jnp.float32),
                   jax.ShapeDtypeStruct((N - NUM_USER, 3 * D), jnp.float32)],
    )(side2, nf, W1, b1.reshape(1, D), W2, b2.reshape(1, D), emb, norm1)


def kernel(adj_indices, adj_values, emb,
           W1_0, b1_0, W2_0, b2_0, W1_1, b1_1, W2_1, b2_1):
    idxf = adj_indices.reshape(2 * E)
    side_p1 = _spmm(idxf, adj_values, emb)
    nf1, norm1 = _dense(side_p1, emb, W1_0, b1_0, W2_0, b2_0)
    side_p2 = _spmm(idxf, adj_values, nf1)
    return _dense_final(side_p2, nf1, W1_1, b1_1, W2_1, b2_1, emb, norm1)
